# baseline (speedup 1.0000x reference)
import functools

import jax
import jax.numpy as jnp
from jax import lax
from jax.experimental import pallas as pl
from jax.experimental.pallas import tpu as pltpu
from jax.experimental.pallas import tpu_sc as plsc

_N = 2_000_000
_NUM_SPECIES = 119
_SCALE = 1.5
_SHIFT = -2.0

_NW = 32
_S = 256_000
_CH = _S // _NW
_LANES = 16

_TC_LANES = 128
_ROWS = _N // _TC_LANES
_SC_ROWS = _S // _TC_LANES
_BLK = 2000
_TC_GRID = -(-(_ROWS - _SC_ROWS) // _BLK)


def _sc_body(en_hbm, spec_hbm, table_hbm, out_hbm, table_v, spec_v, en_v, out_v):
    wid = lax.axis_index("s") * 2 + lax.axis_index("c")
    pltpu.sync_copy(table_hbm, table_v)
    base = wid * _CH
    pltpu.sync_copy(spec_hbm.at[pl.ds(base, _CH)], spec_v)
    pltpu.sync_copy(en_hbm.at[pl.ds(base, _CH)], en_v)

    @plsc.parallel_loop(0, _CH, step=_LANES, unroll=8)
    def vec_body(j):
        sl = pl.ds(j, _LANES)
        idx = spec_v[sl]
        g = plsc.load_gather(table_v, [idx])
        out_v[sl] = g + en_v[sl] * _SCALE + _SHIFT

    pltpu.sync_copy(out_v, out_hbm.at[pl.ds(base, _CH)])


def _tc_body(tab_ref, sp_ref, en_ref, out_ref):
    tab = tab_ref[0:1, :]
    idx = sp_ref[...]
    g = jnp.take_along_axis(jnp.broadcast_to(tab, idx.shape), idx, axis=1)
    out_ref[...] = g + en_ref[...] * _SCALE + _SHIFT


@jax.jit
def _hybrid(per_atom_energies, species, table_flat):
    mesh = plsc.VectorSubcoreMesh(core_axis_name="c", subcore_axis_name="s")
    sc_fn = functools.partial(
        pl.kernel,
        out_type=jax.ShapeDtypeStruct((_S,), jnp.float32),
        mesh=mesh,
        scratch_types=[
            pltpu.VMEM((_NUM_SPECIES,), jnp.float32),
            pltpu.VMEM((_CH,), jnp.int32),
            pltpu.VMEM((_CH,), jnp.float32),
            pltpu.VMEM((_CH,), jnp.float32),
        ],
        compiler_params=pltpu.CompilerParams(needs_layout_passes=False),
    )(_sc_body)
    out_sc = sc_fn(per_atom_energies, species, table_flat)

    sp2 = species.reshape(_ROWS, _TC_LANES)
    en2 = per_atom_energies.reshape(_ROWS, _TC_LANES)
    tab2 = jnp.pad(table_flat, (0, _TC_LANES - _NUM_SPECIES)).reshape(1, _TC_LANES)
    out_tc = pl.pallas_call(
        _tc_body,
        grid=(_TC_GRID,),
        in_specs=[
            pl.BlockSpec((1, _TC_LANES), lambda i: (0, 0)),
            pl.BlockSpec((_BLK, _TC_LANES), lambda i: (i + 1, 0)),
            pl.BlockSpec((_BLK, _TC_LANES), lambda i: (i + 1, 0)),
        ],
        out_specs=pl.BlockSpec((_BLK, _TC_LANES), lambda i: (i + 1, 0)),
        out_shape=jax.ShapeDtypeStruct((_ROWS, _TC_LANES), jnp.float32),
    )(tab2, sp2, en2)
    return lax.dynamic_update_slice(out_tc.reshape(_N), out_sc, (0,))


def kernel(per_atom_energies, species, atomic_energy_table):
    species = species.astype(jnp.int32)
    return _hybrid(per_atom_energies, species,
                   atomic_energy_table.reshape(_NUM_SPECIES))


# speedup vs baseline: 1.0240x; 1.0240x over previous
import functools

import jax
import jax.numpy as jnp
from jax import lax
from jax.experimental import pallas as pl
from jax.experimental.pallas import tpu as pltpu
from jax.experimental.pallas import tpu_sc as plsc

_N = 2_000_000
_NUM_SPECIES = 119
_TABLE_PAD = 128
_SCALE = 1.5
_SHIFT = -2.0

_NW = 32
_S = 256_000
_CH = _S // _NW
_LANES = 16

_TC_LANES = 128
_ROWS = _N // _TC_LANES
_SC_ROWS = _S // _TC_LANES
_BLK = 2000
_TC_GRID = -(-(_ROWS - _SC_ROWS) // _BLK)


def _sc_body(en_hbm, spec_hbm, table_hbm, out_hbm, table_v, spec_v, en_v, out_v):
    wid = lax.axis_index("s") * 2 + lax.axis_index("c")
    pltpu.sync_copy(table_hbm, table_v)
    base = wid * _CH
    pltpu.sync_copy(spec_hbm.at[pl.ds(base, _CH)], spec_v)
    pltpu.sync_copy(en_hbm.at[pl.ds(base, _CH)], en_v)

    @plsc.parallel_loop(0, _CH, step=_LANES, unroll=8)
    def vec_body(j):
        sl = pl.ds(j, _LANES)
        idx = spec_v[sl]
        g = plsc.load_gather(table_v, [idx])
        out_v[sl] = g + en_v[sl] * _SCALE + _SHIFT

    pltpu.sync_copy(out_v, out_hbm.at[pl.ds(base, _CH)])


def _tc_body(tab_ref, sp_ref, en_ref, out_ref):
    tab = tab_ref[0:1, :]
    idx = sp_ref[...]
    g = jnp.take_along_axis(jnp.broadcast_to(tab, idx.shape), idx, axis=1)
    out_ref[...] = g + en_ref[...] * _SCALE + _SHIFT


@jax.jit
def _hybrid(per_atom_energies, species, table_padded):
    mesh = plsc.VectorSubcoreMesh(core_axis_name="c", subcore_axis_name="s")
    sc_fn = functools.partial(
        pl.kernel,
        out_type=jax.ShapeDtypeStruct((_S,), jnp.float32),
        mesh=mesh,
        scratch_types=[
            pltpu.VMEM((_TABLE_PAD,), jnp.float32),
            pltpu.VMEM((_CH,), jnp.int32),
            pltpu.VMEM((_CH,), jnp.float32),
            pltpu.VMEM((_CH,), jnp.float32),
        ],
        compiler_params=pltpu.CompilerParams(needs_layout_passes=False),
    )(_sc_body)
    out_sc = sc_fn(per_atom_energies, species, table_padded)

    sp2 = species.reshape(_ROWS, _TC_LANES)
    en2 = per_atom_energies.reshape(_ROWS, _TC_LANES)
    tab2 = table_padded.reshape(1, _TC_LANES)
    out_tc = pl.pallas_call(
        _tc_body,
        grid=(_TC_GRID,),
        in_specs=[
            pl.BlockSpec((1, _TC_LANES), lambda i: (0, 0)),
            pl.BlockSpec((_BLK, _TC_LANES), lambda i: (i + 1, 0)),
            pl.BlockSpec((_BLK, _TC_LANES), lambda i: (i + 1, 0)),
        ],
        out_specs=pl.BlockSpec((_BLK, _TC_LANES), lambda i: (i + 1, 0)),
        out_shape=jax.ShapeDtypeStruct((_ROWS, _TC_LANES), jnp.float32),
    )(tab2, sp2, en2)
    return lax.dynamic_update_slice(out_tc.reshape(_N), out_sc, (0,))


def kernel(per_atom_energies, species, atomic_energy_table):
    species = species.astype(jnp.int32)
    table = jnp.pad(atomic_energy_table.reshape(-1),
                    (0, _TABLE_PAD - _NUM_SPECIES))
    return _hybrid(per_atom_energies, species, table)
